# Initial kernel scaffold; baseline (speedup 1.0000x reference)
import jax
import jax.numpy as jnp
from jax.experimental import pallas as pl


def kernel(c1_w, c1_b, c2_w, c2_b, f1_w, f1_b, f2_w, f2_b, f3_w, f3_b, x_nchw):
    raise NotImplementedError("write your pallas kernel here")


# speedup vs baseline: 54.8236x; 54.8236x over previous
import numpy as np

import jax
import jax.numpy as jnp
from jax.experimental import pallas as pl
from jax.experimental.pallas import tpu as pltpu


def _round_up(x, m):
    return (x + m - 1) // m * m


def _conv1_toeplitz(c1_w):
    r = np.arange(6)[:, None, None]
    i = np.arange(96)[None, :, None]
    j = np.arange(512)[None, None, :]
    t = j // 128
    dy, dx = t // 2, t % 2
    jj = j % 128
    owp, cout = jj // 6, jj % 6
    w, cin = i // 3, i % 3
    kh = r - dy
    kw = w - 2 * owp - dx
    valid = (jj < 84) & (kh >= 0) & (kh < 5) & (kw >= 0) & (kw < 5)
    flat = np.where(valid, ((kh * 5 + kw) * 3 + cin) * 128 + cout, 0)
    vals = c1_w.reshape(-1)[jnp.asarray(flat.reshape(-1))].reshape(6, 96, 512)
    return jnp.where(jnp.asarray(valid), vals, 0.0)


def _conv2_toeplitz(c2_w):
    r = np.arange(6)[:, None, None]
    i = np.arange(128)[None, :, None]
    j = np.arange(512)[None, None, :]
    t = j // 128
    dy, dx = t // 2, t % 2
    jj = j % 128
    owp, cout = jj // 16, jj % 16
    w, cin = i // 6, i % 6
    kh = r - dy
    kw = w - 2 * owp - dx
    valid = ((jj < 80) & (i < 84) & (kh >= 0) & (kh < 5)
             & (kw >= 0) & (kw < 5))
    flat = np.where(valid, ((kh * 5 + kw) * 6 + cin) * 128 + cout, 0)
    vals = c2_w.reshape(-1)[jnp.asarray(flat.reshape(-1))].reshape(6, 128, 512)
    return jnp.where(jnp.asarray(valid), vals, 0.0)


def _lenet_kernel(x_ref, w1_ref, b1_ref, w2_ref, b2_ref, f1_ref, fb1_ref,
                  f2_ref, fb2_ref, f3_ref, fb3_ref, o_ref):
    x = x_ref[...]
    b1 = b1_ref[...]
    b2 = b2_ref[...]

    p1 = []
    for ohp in range(14):
        acc = jnp.dot(x[:, 2 * ohp, :], w1_ref[0],
                      preferred_element_type=jnp.float32)
        for rr in range(1, 6):
            acc = acc + jnp.dot(x[:, 2 * ohp + rr, :], w1_ref[rr],
                                preferred_element_type=jnp.float32)
        m = jnp.maximum(jnp.maximum(acc[:, 0:128], acc[:, 128:256]),
                        jnp.maximum(acc[:, 256:384], acc[:, 384:512]))
        p1.append(jnp.maximum(m + b1, 0.0))

    p2 = []
    for ohp in range(5):
        acc = jnp.dot(p1[2 * ohp], w2_ref[0],
                      preferred_element_type=jnp.float32)
        for rr in range(1, 6):
            acc = acc + jnp.dot(p1[2 * ohp + rr], w2_ref[rr],
                                preferred_element_type=jnp.float32)
        m = jnp.maximum(jnp.maximum(acc[:, 0:128], acc[:, 128:256]),
                        jnp.maximum(acc[:, 256:384], acc[:, 384:512]))
        p2.append(jnp.maximum(m + b2, 0.0))

    h = jnp.dot(p2[0], f1_ref[0], preferred_element_type=jnp.float32)
    for hh in range(1, 5):
        h = h + jnp.dot(p2[hh], f1_ref[hh],
                        preferred_element_type=jnp.float32)
    h = jnp.maximum(h + fb1_ref[...], 0.0)
    h = jnp.dot(h, f2_ref[...], preferred_element_type=jnp.float32)
    h = jnp.maximum(h + fb2_ref[...], 0.0)
    h = jnp.dot(h, f3_ref[...], preferred_element_type=jnp.float32)
    o_ref[...] = (h + fb3_ref[...]).astype(o_ref.dtype)


def kernel(c1_w, c1_b, c2_w, c2_b, f1_w, f1_b, f2_w, f2_b, f3_w, f3_b,
           x_nchw):
    B = x_nchw.shape[0]

    xr = jnp.transpose(x_nchw, (0, 2, 3, 1)).reshape(B, 32, 96)

    w1 = _conv1_toeplitz(c1_w)
    w2 = _conv2_toeplitz(c2_w)
    b1p = jnp.concatenate(
        [jnp.tile(c1_b[0, :6], 14), jnp.zeros(44, c1_b.dtype)]).reshape(1, 128)
    b2p = jnp.concatenate(
        [jnp.tile(c2_b[0, :16], 5), jnp.zeros(48, c2_b.dtype)]).reshape(1, 128)
    f1r = jnp.pad(f1_w.reshape(5, 80, 128), ((0, 0), (0, 48), (0, 0)))

    bt = min(256, _round_up(B, 8))
    Bp = _round_up(B, bt)
    if Bp != B:
        xr = jnp.pad(xr, ((0, Bp - B), (0, 0), (0, 0)))

    out = pl.pallas_call(
        _lenet_kernel,
        out_shape=jax.ShapeDtypeStruct((Bp, 128), jnp.float32),
        grid=(Bp // bt,),
        in_specs=[
            pl.BlockSpec((bt, 32, 96), lambda m: (m, 0, 0)),
            pl.BlockSpec((6, 96, 512), lambda m: (0, 0, 0)),
            pl.BlockSpec((1, 128), lambda m: (0, 0)),
            pl.BlockSpec((6, 128, 512), lambda m: (0, 0, 0)),
            pl.BlockSpec((1, 128), lambda m: (0, 0)),
            pl.BlockSpec((5, 128, 128), lambda m: (0, 0, 0)),
            pl.BlockSpec((1, 128), lambda m: (0, 0)),
            pl.BlockSpec((128, 128), lambda m: (0, 0)),
            pl.BlockSpec((1, 128), lambda m: (0, 0)),
            pl.BlockSpec((128, 128), lambda m: (0, 0)),
            pl.BlockSpec((1, 128), lambda m: (0, 0)),
        ],
        out_specs=pl.BlockSpec((bt, 128), lambda m: (m, 0)),
        compiler_params=pltpu.CompilerParams(
            dimension_semantics=("parallel",)),
    )(xr, w1, b1p, w2, b2p, f1r, f1_b, f2_w, f2_b, f3_w, f3_b)
    return out[:B, :10]


# speedup vs baseline: 1282.7602x; 23.3979x over previous
import numpy as np

import jax
import jax.numpy as jnp
from jax.experimental import pallas as pl
from jax.experimental.pallas import tpu as pltpu


def _round_up(x, m):
    return (x + m - 1) // m * m


def _toeplitz(w_lane, n_rows, k, cin, cout, owp_n):
    kk = k * k * cin
    r = np.arange(6)[:, None, None, None]
    i = np.arange(n_rows)[None, :, None, None]
    t = np.arange(4)[None, None, :, None]
    owp = np.arange(owp_n)[None, None, None, :]
    dy, dx = t // 2, t % 2
    w, ci = i // cin, i % cin
    kh = r - dy
    kw = w - 2 * owp - dx
    valid = (kh >= 0) & (kh < k) & (kw >= 0) & (kw < k)
    rows = np.where(valid, (kh * k + kw) * cin + ci, -1)
    rows_dev = jnp.asarray(rows.reshape(-1, 1), jnp.int32)
    onehot = (rows_dev == jnp.arange(kk, dtype=jnp.int32)).astype(w_lane.dtype)
    blk = jax.lax.dot_general(
        onehot, w_lane,
        (((1,), (0,)), ((), ())))
    blk = blk.reshape(6, n_rows, 4, owp_n * cout)
    blk = jnp.pad(blk, ((0, 0), (0, 0), (0, 0), (0, 128 - owp_n * cout)))
    return blk.reshape(6, n_rows, 512)


def _interleave_mats():
    e = np.zeros((3, 32, 128), np.float32)
    for c in range(3):
        e[c, np.arange(32), np.arange(32) * 3 + c] = 1.0
    return jnp.asarray(e)


def _lenet_kernel(x_ref, e_ref, w1_ref, b1_ref, w2_ref, b2_ref, f1_ref,
                  fb1_ref, f2_ref, fb2_ref, f3_ref, fb3_ref, o_ref):
    bt = x_ref.shape[0]
    b1 = b1_ref[...]
    b2 = b2_ref[...]

    x = x_ref[...].astype(jnp.bfloat16)
    xr = jnp.dot(x[:, 0:32, :].reshape(bt * 32, 32), e_ref[0],
                 preferred_element_type=jnp.float32)
    xr = xr + jnp.dot(x[:, 32:64, :].reshape(bt * 32, 32), e_ref[1],
                      preferred_element_type=jnp.float32)
    xr = xr + jnp.dot(x[:, 64:96, :].reshape(bt * 32, 32), e_ref[2],
                      preferred_element_type=jnp.float32)
    xr = xr.astype(jnp.bfloat16).reshape(bt, 32, 128)

    p1 = []
    for ohp in range(14):
        acc = jnp.dot(xr[:, 2 * ohp, :], w1_ref[0],
                      preferred_element_type=jnp.float32)
        for rr in range(1, 6):
            acc = acc + jnp.dot(xr[:, 2 * ohp + rr, :], w1_ref[rr],
                                preferred_element_type=jnp.float32)
        m = jnp.maximum(jnp.maximum(acc[:, 0:128], acc[:, 128:256]),
                        jnp.maximum(acc[:, 256:384], acc[:, 384:512]))
        p1.append(jnp.maximum(m + b1, 0.0).astype(jnp.bfloat16))

    p2 = []
    for ohp in range(5):
        acc = jnp.dot(p1[2 * ohp], w2_ref[0],
                      preferred_element_type=jnp.float32)
        for rr in range(1, 6):
            acc = acc + jnp.dot(p1[2 * ohp + rr], w2_ref[rr],
                                preferred_element_type=jnp.float32)
        m = jnp.maximum(jnp.maximum(acc[:, 0:128], acc[:, 128:256]),
                        jnp.maximum(acc[:, 256:384], acc[:, 384:512]))
        p2.append(jnp.maximum(m + b2, 0.0).astype(jnp.bfloat16))

    h = jnp.dot(p2[0], f1_ref[0], preferred_element_type=jnp.float32)
    for hh in range(1, 5):
        h = h + jnp.dot(p2[hh], f1_ref[hh],
                        preferred_element_type=jnp.float32)
    h = jnp.maximum(h + fb1_ref[...], 0.0).astype(jnp.bfloat16)
    h = jnp.dot(h, f2_ref[...], preferred_element_type=jnp.float32)
    h = jnp.maximum(h + fb2_ref[...], 0.0).astype(jnp.bfloat16)
    h = jnp.dot(h, f3_ref[...], preferred_element_type=jnp.float32)
    o_ref[...] = (h + fb3_ref[...]).astype(o_ref.dtype)


def kernel(c1_w, c1_b, c2_w, c2_b, f1_w, f1_b, f2_w, f2_b, f3_w, f3_b,
           x_nchw):
    B = x_nchw.shape[0]

    xq = x_nchw.reshape(B, 96, 32)

    ee = _interleave_mats().astype(jnp.bfloat16)
    w1 = _toeplitz(c1_w[:, :6], 128, 5, 3, 6, 14).astype(jnp.bfloat16)
    w2 = _toeplitz(c2_w[:, :16], 128, 5, 6, 16, 5).astype(jnp.bfloat16)
    b1p = jnp.concatenate(
        [jnp.tile(c1_b[0, :6], 14), jnp.zeros(44, c1_b.dtype)]).reshape(1, 128)
    b2p = jnp.concatenate(
        [jnp.tile(c2_b[0, :16], 5), jnp.zeros(48, c2_b.dtype)]).reshape(1, 128)
    f1r = jnp.pad(f1_w.reshape(5, 80, 128),
                  ((0, 0), (0, 48), (0, 0))).astype(jnp.bfloat16)
    f2b16 = f2_w.astype(jnp.bfloat16)
    f3b16 = f3_w.astype(jnp.bfloat16)

    bt = min(256, _round_up(B, 8))
    Bp = _round_up(B, bt)
    if Bp != B:
        xq = jnp.pad(xq, ((0, Bp - B), (0, 0), (0, 0)))

    out = pl.pallas_call(
        _lenet_kernel,
        out_shape=jax.ShapeDtypeStruct((Bp, 128), jnp.float32),
        grid=(Bp // bt,),
        in_specs=[
            pl.BlockSpec((bt, 96, 32), lambda m: (m, 0, 0)),
            pl.BlockSpec((3, 32, 128), lambda m: (0, 0, 0)),
            pl.BlockSpec((6, 128, 512), lambda m: (0, 0, 0)),
            pl.BlockSpec((1, 128), lambda m: (0, 0)),
            pl.BlockSpec((6, 128, 512), lambda m: (0, 0, 0)),
            pl.BlockSpec((1, 128), lambda m: (0, 0)),
            pl.BlockSpec((5, 128, 128), lambda m: (0, 0, 0)),
            pl.BlockSpec((1, 128), lambda m: (0, 0)),
            pl.BlockSpec((128, 128), lambda m: (0, 0)),
            pl.BlockSpec((1, 128), lambda m: (0, 0)),
            pl.BlockSpec((128, 128), lambda m: (0, 0)),
            pl.BlockSpec((1, 128), lambda m: (0, 0)),
        ],
        out_specs=pl.BlockSpec((bt, 128), lambda m: (m, 0)),
        compiler_params=pltpu.CompilerParams(
            dimension_semantics=("parallel",)),
    )(xq, ee, w1, b1p, w2, b2p, f1r, f1_b, f2b16, f2_b, f3b16, f3_b)
    return out[:B, :10]


# speedup vs baseline: 1444.2667x; 1.1259x over previous
import numpy as np

import jax
import jax.numpy as jnp
from jax.experimental import pallas as pl
from jax.experimental.pallas import tpu as pltpu


def _round_up(x, m):
    return (x + m - 1) // m * m


def _toeplitz(w_lane, n_rows, k, cin, cout, owp_n):
    kk = k * k * cin
    r = np.arange(6)[:, None, None, None]
    i = np.arange(n_rows)[None, :, None, None]
    t = np.arange(4)[None, None, :, None]
    owp = np.arange(owp_n)[None, None, None, :]
    dy, dx = t // 2, t % 2
    w, ci = i // cin, i % cin
    kh = r - dy
    kw = w - 2 * owp - dx
    valid = (kh >= 0) & (kh < k) & (kw >= 0) & (kw < k)
    rows = np.where(valid, (kh * k + kw) * cin + ci, -1)
    rows_dev = jnp.asarray(rows.reshape(-1, 1), jnp.int32)
    onehot = (rows_dev == jnp.arange(kk, dtype=jnp.int32)).astype(w_lane.dtype)
    blk = jax.lax.dot_general(
        onehot, w_lane,
        (((1,), (0,)), ((), ())))
    blk = blk.reshape(6, n_rows, 4, owp_n * cout)
    blk = jnp.pad(blk, ((0, 0), (0, 0), (0, 0), (0, 128 - owp_n * cout)))
    return blk.reshape(6, n_rows, 512)


def _interleave_mats():
    e = np.zeros((3, 128, 512), np.float32)
    hm = np.repeat(np.arange(4), 32)
    w = np.tile(np.arange(32), 4)
    for c in range(3):
        e[c, np.arange(128), hm * 128 + w * 3 + c] = 1.0
    return jnp.asarray(e)


def _lenet_kernel(x_ref, e_ref, w1_ref, b1_ref, w2_ref, b2_ref, f1_ref,
                  fb1_ref, f2_ref, fb2_ref, f3_ref, fb3_ref, o_ref):
    bt = x_ref.shape[0]
    b1 = b1_ref[...]
    b2 = b2_ref[...]

    x = x_ref[...].astype(jnp.bfloat16)
    xr = jnp.dot(x[:, 0:8, :].reshape(bt * 8, 128), e_ref[0],
                 preferred_element_type=jnp.float32)
    xr = xr + jnp.dot(x[:, 8:16, :].reshape(bt * 8, 128), e_ref[1],
                      preferred_element_type=jnp.float32)
    xr = xr + jnp.dot(x[:, 16:24, :].reshape(bt * 8, 128), e_ref[2],
                      preferred_element_type=jnp.float32)
    xr = xr.astype(jnp.bfloat16).reshape(bt, 8, 512)
    grp = [xr[:, g, :] for g in range(8)]
    rows = [grp[hh // 4][:, (hh % 4) * 128:(hh % 4) * 128 + 128]
            for hh in range(32)]

    p1 = []
    for ohp in range(14):
        acc = jnp.dot(rows[2 * ohp], w1_ref[0],
                      preferred_element_type=jnp.float32)
        for rr in range(1, 6):
            acc = acc + jnp.dot(rows[2 * ohp + rr], w1_ref[rr],
                                preferred_element_type=jnp.float32)
        m = jnp.maximum(jnp.maximum(acc[:, 0:128], acc[:, 128:256]),
                        jnp.maximum(acc[:, 256:384], acc[:, 384:512]))
        p1.append(jnp.maximum(m + b1, 0.0).astype(jnp.bfloat16))

    p2 = []
    for ohp in range(5):
        acc = jnp.dot(p1[2 * ohp], w2_ref[0],
                      preferred_element_type=jnp.float32)
        for rr in range(1, 6):
            acc = acc + jnp.dot(p1[2 * ohp + rr], w2_ref[rr],
                                preferred_element_type=jnp.float32)
        m = jnp.maximum(jnp.maximum(acc[:, 0:128], acc[:, 128:256]),
                        jnp.maximum(acc[:, 256:384], acc[:, 384:512]))
        p2.append(jnp.maximum(m + b2, 0.0).astype(jnp.bfloat16))

    h = jnp.dot(p2[0], f1_ref[0], preferred_element_type=jnp.float32)
    for hh in range(1, 5):
        h = h + jnp.dot(p2[hh], f1_ref[hh],
                        preferred_element_type=jnp.float32)
    h = jnp.maximum(h + fb1_ref[...], 0.0).astype(jnp.bfloat16)
    h = jnp.dot(h, f2_ref[...], preferred_element_type=jnp.float32)
    h = jnp.maximum(h + fb2_ref[...], 0.0).astype(jnp.bfloat16)
    h = jnp.dot(h, f3_ref[...], preferred_element_type=jnp.float32)
    o_ref[...] = (h + fb3_ref[...]).astype(o_ref.dtype)


def kernel(c1_w, c1_b, c2_w, c2_b, f1_w, f1_b, f2_w, f2_b, f3_w, f3_b,
           x_nchw):
    B = x_nchw.shape[0]

    xq = x_nchw.reshape(B, 24, 128)

    ee = _interleave_mats().astype(jnp.bfloat16)
    w1 = _toeplitz(c1_w[:, :6], 128, 5, 3, 6, 14).astype(jnp.bfloat16)
    w2 = _toeplitz(c2_w[:, :16], 128, 5, 6, 16, 5).astype(jnp.bfloat16)
    b1p = jnp.concatenate(
        [jnp.tile(c1_b[0, :6], 14), jnp.zeros(44, c1_b.dtype)]).reshape(1, 128)
    b2p = jnp.concatenate(
        [jnp.tile(c2_b[0, :16], 5), jnp.zeros(48, c2_b.dtype)]).reshape(1, 128)
    f1r = jnp.pad(f1_w.reshape(5, 80, 128),
                  ((0, 0), (0, 48), (0, 0))).astype(jnp.bfloat16)
    f2b16 = f2_w.astype(jnp.bfloat16)
    f3b16 = f3_w.astype(jnp.bfloat16)

    bt = min(256, _round_up(B, 8))
    Bp = _round_up(B, bt)
    if Bp != B:
        xq = jnp.pad(xq, ((0, Bp - B), (0, 0), (0, 0)))

    out = pl.pallas_call(
        _lenet_kernel,
        out_shape=jax.ShapeDtypeStruct((Bp, 128), jnp.float32),
        grid=(Bp // bt,),
        in_specs=[
            pl.BlockSpec((bt, 24, 128), lambda m: (m, 0, 0)),
            pl.BlockSpec((3, 128, 512), lambda m: (0, 0, 0)),
            pl.BlockSpec((6, 128, 512), lambda m: (0, 0, 0)),
            pl.BlockSpec((1, 128), lambda m: (0, 0)),
            pl.BlockSpec((6, 128, 512), lambda m: (0, 0, 0)),
            pl.BlockSpec((1, 128), lambda m: (0, 0)),
            pl.BlockSpec((5, 128, 128), lambda m: (0, 0, 0)),
            pl.BlockSpec((1, 128), lambda m: (0, 0)),
            pl.BlockSpec((128, 128), lambda m: (0, 0)),
            pl.BlockSpec((1, 128), lambda m: (0, 0)),
            pl.BlockSpec((128, 128), lambda m: (0, 0)),
            pl.BlockSpec((1, 128), lambda m: (0, 0)),
        ],
        out_specs=pl.BlockSpec((bt, 128), lambda m: (m, 0)),
        compiler_params=pltpu.CompilerParams(
            dimension_semantics=("parallel",)),
    )(xq, ee, w1, b1p, w2, b2p, f1r, f1_b, f2b16, f2_b, f3b16, f3_b)
    return out[:B, :10]
